# baseline (device time: 188797 ns/iter reference)
import jax
import jax.numpy as jnp
from jax import lax
from jax.experimental import pallas as pl
from jax.experimental.pallas import tpu as pltpu

N_DEV = 16
SUB = 4


def kernel(x, w_mat):
    m_per, k = x.shape
    _, n_per = w_mat.shape
    half = m_per // 2
    subrows = half // SUB

    def body(x_ref, w_ref, out_ref, cw_ref, ccw_ref, wb_ref, amax_ref,
             cw_send_sems, cw_recv_sems, ccw_send_sems, ccw_recv_sems,
             a_send_sems, a_recv_sems):
        my = lax.axis_index("i")
        left = lax.rem(my - 1 + N_DEV, N_DEV)
        right = lax.rem(my + 1, N_DEV)

        barrier_sem = pltpu.get_barrier_semaphore()
        for nbr in (left, right):
            pl.semaphore_signal(
                barrier_sem, inc=1,
                device_id=(nbr,), device_id_type=pl.DeviceIdType.MESH,
            )
        pl.semaphore_wait(barrier_sem, 2)

        wb_ref[...] = w_ref[...].astype(jnp.bfloat16)
        xb = x_ref[...].astype(jnp.bfloat16)
        cw_ref[0] = xb[:half, :].reshape(SUB, subrows, k)
        ccw_ref[0] = xb[half:, :].reshape(SUB, subrows, k)

        def make(h, j):
            s, r = h % 2, (h + 1) % 2
            cw = pltpu.make_async_remote_copy(
                src_ref=cw_ref.at[s, j], dst_ref=cw_ref.at[r, j],
                send_sem=cw_send_sems.at[s, j], recv_sem=cw_recv_sems.at[r, j],
                device_id=(right,), device_id_type=pl.DeviceIdType.MESH,
            )
            ccw = pltpu.make_async_remote_copy(
                src_ref=ccw_ref.at[s, j], dst_ref=ccw_ref.at[r, j],
                send_sem=ccw_send_sems.at[s, j], recv_sem=ccw_recv_sems.at[r, j],
                device_id=(left,), device_id_type=pl.DeviceIdType.MESH,
            )
            return cw, ccw

        hops = [[make(h, j) for j in range(SUB)] for h in range(N_DEV - 1)]
        for j in range(SUB):
            hops[0][j][0].start()
            hops[0][j][1].start()

        def gemm(chunk, row0, rows):
            y = lax.dot_general(
                chunk, wb_ref[...], (((1,), (0,)), ((), ())),
                preferred_element_type=jnp.float32,
            )
            y = jnp.maximum(y, 0.0)
            out_ref[pl.ds(row0, rows), :] = y
            return jnp.max(y)

        local_amax = gemm(xb, my * m_per, m_per)

        for h in range(N_DEV - 1):
            for j in range(SUB):
                cw, ccw = hops[h][j]
                cw.wait_recv()
                ccw.wait_recv()
                if h + 1 < N_DEV - 1:
                    if h >= 1:
                        hops[h - 1][j][0].wait_send()
                        hops[h - 1][j][1].wait_send()
                    hops[h + 1][j][0].start()
                    hops[h + 1][j][1].start()
            r = (h + 1) % 2
            ocw = lax.rem(my - (h + 1) + N_DEV, N_DEV)
            occw = lax.rem(my + (h + 1), N_DEV)
            for j in range(SUB):
                a1 = gemm(cw_ref[r, j], ocw * m_per + j * subrows, subrows)
                a2 = gemm(ccw_ref[r, j],
                          occw * m_per + half + j * subrows, subrows)
                local_amax = jnp.maximum(local_amax, jnp.maximum(a1, a2))

        for h in (N_DEV - 3, N_DEV - 2):
            for j in range(SUB):
                hops[h][j][0].wait_send()
                hops[h][j][1].wait_send()

        amax_ref[N_DEV - 1, :, :] = jnp.full((8, 128), local_amax, jnp.float32)
        sends = []
        for d in range(1, N_DEV):
            tgt = lax.rem(my + d, N_DEV)
            c = pltpu.make_async_remote_copy(
                src_ref=amax_ref.at[N_DEV - 1],
                dst_ref=amax_ref.at[d - 1],
                send_sem=a_send_sems.at[d - 1],
                recv_sem=a_recv_sems.at[d - 1],
                device_id=(tgt,),
                device_id_type=pl.DeviceIdType.MESH,
            )
            c.start()
            sends.append(c)
        g = local_amax
        for d, c in zip(range(1, N_DEV), sends):
            c.wait_send()
            c.wait_recv()
            g = jnp.maximum(g, amax_ref[d - 1, 0, 0])

        scale = g / 127.0
        q = jnp.clip(jnp.round(out_ref[...] / scale), -127.0, 127.0)
        out_ref[...] = q * scale

    return pl.pallas_call(
        body,
        out_shape=jax.ShapeDtypeStruct((N_DEV * m_per, n_per), jnp.float32),
        in_specs=[
            pl.BlockSpec(memory_space=pltpu.VMEM),
            pl.BlockSpec(memory_space=pltpu.VMEM),
        ],
        out_specs=pl.BlockSpec(memory_space=pltpu.VMEM),
        scratch_shapes=[
            pltpu.VMEM((2, SUB, subrows, k), jnp.bfloat16),
            pltpu.VMEM((2, SUB, subrows, k), jnp.bfloat16),
            pltpu.VMEM((k, n_per), jnp.bfloat16),
            pltpu.VMEM((N_DEV, 8, 128), jnp.float32),
            pltpu.SemaphoreType.DMA((2, SUB)),
            pltpu.SemaphoreType.DMA((2, SUB)),
            pltpu.SemaphoreType.DMA((2, SUB)),
            pltpu.SemaphoreType.DMA((2, SUB)),
            pltpu.SemaphoreType.DMA((N_DEV - 1,)),
            pltpu.SemaphoreType.DMA((N_DEV - 1,)),
        ],
        compiler_params=pltpu.CompilerParams(collective_id=0),
    )(x, w_mat)


# device time: 164650 ns/iter; 1.1467x vs baseline; 1.1467x over previous
import jax
import jax.numpy as jnp
from jax import lax
from jax.experimental import pallas as pl
from jax.experimental.pallas import tpu as pltpu

N_DEV = 16
SUB = 2
HALF = 128
SUBR = HALF // SUB


def kernel(x, w_mat):
    m_per, k = x.shape
    _, n_per = w_mat.shape

    def body(x_ref, w_ref, out_ref, colbuf, cwbuf, ccwbuf, wb_ref, amax_ref,
             a1_send, a1_recv, cw_send, cw_recv, ccw_send, ccw_recv,
             ax_send, ax_recv):
        my = lax.axis_index("i")
        r = lax.rem(my, 4)
        base = my - r
        up = lax.rem(my + 4, N_DEV)
        down = lax.rem(my + 12, N_DEV)
        pright = base + lax.rem(r + 1, 4)
        pleft = base + lax.rem(r + 3, 4)

        barrier_sem = pltpu.get_barrier_semaphore()
        for nbr in (up, down, pright, pleft):
            pl.semaphore_signal(
                barrier_sem, inc=1,
                device_id=(nbr,), device_id_type=pl.DeviceIdType.MESH,
            )
        pl.semaphore_wait(barrier_sem, 4)

        wb_ref[...] = w_ref[...].astype(jnp.bfloat16)
        xb = x_ref[...].astype(jnp.bfloat16)
        colbuf[3] = xb

        cw1 = pltpu.make_async_remote_copy(
            src_ref=colbuf.at[3], dst_ref=colbuf.at[0],
            send_sem=a1_send.at[0], recv_sem=a1_recv.at[0],
            device_id=(up,), device_id_type=pl.DeviceIdType.MESH,
        )
        cw2 = pltpu.make_async_remote_copy(
            src_ref=colbuf.at[0], dst_ref=colbuf.at[1],
            send_sem=a1_send.at[1], recv_sem=a1_recv.at[1],
            device_id=(up,), device_id_type=pl.DeviceIdType.MESH,
        )
        ccw1 = pltpu.make_async_remote_copy(
            src_ref=colbuf.at[3], dst_ref=colbuf.at[2],
            send_sem=a1_send.at[2], recv_sem=a1_recv.at[2],
            device_id=(down,), device_id_type=pl.DeviceIdType.MESH,
        )
        cw1.start()
        ccw1.start()

        def gemm(chunk, row0, rows):
            y = lax.dot_general(
                chunk, wb_ref[...], (((1,), (0,)), ((), ())),
                preferred_element_type=jnp.float32,
            )
            y = jnp.maximum(y, 0.0)
            out_ref[pl.ds(row0, rows), :] = y
            return jnp.max(y)

        local_amax = gemm(xb, my * m_per, m_per)

        items = [(3, 0), (0, -1), (2, 1), (1, -2)]

        for i, (slot, dz) in enumerate(items):
            if i == 1:
                cw1.wait_recv()
                cw2.start()
                od = lax.rem(my + dz * 4 + N_DEV, N_DEV)
                local_amax = jnp.maximum(
                    local_amax, gemm(colbuf[slot], od * m_per, m_per))
            elif i == 2:
                ccw1.wait_recv()
                od = lax.rem(my + dz * 4 + N_DEV, N_DEV)
                local_amax = jnp.maximum(
                    local_amax, gemm(colbuf[slot], od * m_per, m_per))
            elif i == 3:
                cw2.wait_recv()
                od = lax.rem(my + dz * 4 + 2 * N_DEV, N_DEV)
                local_amax = jnp.maximum(
                    local_amax, gemm(colbuf[slot], od * m_per, m_per))

            def mk(h, j):
                if h == 0:
                    cw_src = colbuf.at[slot, j * SUBR:(j + 1) * SUBR, :]
                    ccw_src = colbuf.at[
                        slot, HALF + j * SUBR:HALF + (j + 1) * SUBR, :]
                else:
                    s = 1 if h == 1 else 0
                    cw_src = cwbuf.at[i, s, j]
                    ccw_src = ccwbuf.at[i, s, j]
                d = 1 if h in (0, 2) else 0
                cw = pltpu.make_async_remote_copy(
                    src_ref=cw_src, dst_ref=cwbuf.at[i, d, j],
                    send_sem=cw_send.at[i, h, j],
                    recv_sem=cw_recv.at[i, h, j],
                    device_id=(pright,), device_id_type=pl.DeviceIdType.MESH,
                )
                ccw = pltpu.make_async_remote_copy(
                    src_ref=ccw_src, dst_ref=ccwbuf.at[i, d, j],
                    send_sem=ccw_send.at[i, h, j],
                    recv_sem=ccw_recv.at[i, h, j],
                    device_id=(pleft,), device_id_type=pl.DeviceIdType.MESH,
                )
                return cw, ccw

            hops = [[mk(h, j) for j in range(SUB)] for h in range(3)]
            for j in range(SUB):
                hops[0][j][0].start()
                hops[0][j][1].start()
            for h in range(3):
                for j in range(SUB):
                    cw, ccw = hops[h][j]
                    cw.wait_recv()
                    ccw.wait_recv()
                    if h + 1 < 3:
                        hops[h + 1][j][0].start()
                        hops[h + 1][j][1].start()
                d = 1 if h in (0, 2) else 0
                ocw = base + lax.rem(r - h - 1 + 8, 4)
                occw = base + lax.rem(r + h + 1, 4)
                ocw = lax.rem(ocw + dz * 4 + 2 * N_DEV, N_DEV)
                occw = lax.rem(occw + dz * 4 + 2 * N_DEV, N_DEV)
                for j in range(SUB):
                    a1 = gemm(cwbuf[i, d, j],
                              ocw * m_per + j * SUBR, SUBR)
                    a2 = gemm(ccwbuf[i, d, j],
                              occw * m_per + HALF + j * SUBR, SUBR)
                    local_amax = jnp.maximum(local_amax, jnp.maximum(a1, a2))
            for h in range(3):
                for j in range(SUB):
                    hops[h][j][0].wait_send()
                    hops[h][j][1].wait_send()

        cw1.wait_send()
        cw2.wait_send()
        ccw1.wait_send()

        amax_ref[N_DEV - 1, :, :] = jnp.full((8, 128), local_amax, jnp.float32)
        sends = []
        for d in range(1, N_DEV):
            tgt = lax.rem(my + d, N_DEV)
            c = pltpu.make_async_remote_copy(
                src_ref=amax_ref.at[N_DEV - 1],
                dst_ref=amax_ref.at[d - 1],
                send_sem=ax_send.at[d - 1],
                recv_sem=ax_recv.at[d - 1],
                device_id=(tgt,),
                device_id_type=pl.DeviceIdType.MESH,
            )
            c.start()
            sends.append(c)
        g = local_amax
        for d, c in zip(range(1, N_DEV), sends):
            c.wait_send()
            c.wait_recv()
            g = jnp.maximum(g, amax_ref[d - 1, 0, 0])

        scale = g / 127.0
        q = jnp.clip(jnp.round(out_ref[...] / scale), -127.0, 127.0)
        out_ref[...] = q * scale

    return pl.pallas_call(
        body,
        out_shape=jax.ShapeDtypeStruct((N_DEV * m_per, n_per), jnp.float32),
        in_specs=[
            pl.BlockSpec(memory_space=pltpu.VMEM),
            pl.BlockSpec(memory_space=pltpu.VMEM),
        ],
        out_specs=pl.BlockSpec(memory_space=pltpu.VMEM),
        scratch_shapes=[
            pltpu.VMEM((4, m_per, k), jnp.bfloat16),
            pltpu.VMEM((4, 2, SUB, SUBR, k), jnp.bfloat16),
            pltpu.VMEM((4, 2, SUB, SUBR, k), jnp.bfloat16),
            pltpu.VMEM((k, n_per), jnp.bfloat16),
            pltpu.VMEM((N_DEV, 8, 128), jnp.float32),
            pltpu.SemaphoreType.DMA((3,)),
            pltpu.SemaphoreType.DMA((3,)),
            pltpu.SemaphoreType.DMA((4, 3, SUB)),
            pltpu.SemaphoreType.DMA((4, 3, SUB)),
            pltpu.SemaphoreType.DMA((4, 3, SUB)),
            pltpu.SemaphoreType.DMA((4, 3, SUB)),
            pltpu.SemaphoreType.DMA((N_DEV - 1,)),
            pltpu.SemaphoreType.DMA((N_DEV - 1,)),
        ],
        compiler_params=pltpu.CompilerParams(collective_id=0),
    )(x, w_mat)


# device time: 154354 ns/iter; 1.2231x vs baseline; 1.0667x over previous
import jax
import jax.numpy as jnp
from jax import lax
from jax.experimental import pallas as pl
from jax.experimental.pallas import tpu as pltpu

N_DEV = 16
SUB = 2
HALF = 128
SUBR = HALF // SUB


def kernel(x, w_mat):
    m_per, k = x.shape
    _, n_per = w_mat.shape

    def body(x_ref, w_ref, out_ref, colbuf, cwbuf, ccwbuf, wb_ref, amax_ref,
             a1_send, a1_recv, cw_send, cw_recv, ccw_send, ccw_recv,
             ax_send, ax_recv):
        my = lax.axis_index("i")
        r = lax.rem(my, 4)
        base = my - r
        up = lax.rem(my + 4, N_DEV)
        down = lax.rem(my + 12, N_DEV)
        pright = base + lax.rem(r + 1, 4)
        pleft = base + lax.rem(r + 3, 4)

        barrier_sem = pltpu.get_barrier_semaphore()
        for nbr in (up, down, pright, pleft):
            pl.semaphore_signal(
                barrier_sem, inc=1,
                device_id=(nbr,), device_id_type=pl.DeviceIdType.MESH,
            )
        pl.semaphore_wait(barrier_sem, 4)

        wb_ref[...] = w_ref[...].astype(jnp.bfloat16)
        xb = x_ref[...].astype(jnp.bfloat16)
        colbuf[3] = xb

        cw1 = pltpu.make_async_remote_copy(
            src_ref=colbuf.at[3], dst_ref=colbuf.at[0],
            send_sem=a1_send.at[0], recv_sem=a1_recv.at[0],
            device_id=(up,), device_id_type=pl.DeviceIdType.MESH,
        )
        cw2 = pltpu.make_async_remote_copy(
            src_ref=colbuf.at[0], dst_ref=colbuf.at[1],
            send_sem=a1_send.at[1], recv_sem=a1_recv.at[1],
            device_id=(up,), device_id_type=pl.DeviceIdType.MESH,
        )
        ccw1 = pltpu.make_async_remote_copy(
            src_ref=colbuf.at[3], dst_ref=colbuf.at[2],
            send_sem=a1_send.at[2], recv_sem=a1_recv.at[2],
            device_id=(down,), device_id_type=pl.DeviceIdType.MESH,
        )
        cw1.start()
        ccw1.start()

        def gemm(chunk, row0, rows):
            y = lax.dot_general(
                chunk, wb_ref[...], (((1,), (0,)), ((), ())),
                preferred_element_type=jnp.float32,
            )
            y = jnp.maximum(y, 0.0)
            out_ref[pl.ds(row0, rows), :] = y
            return jnp.max(y)

        local_amax = gemm(xb, my * m_per, m_per)

        items = [(3, 0), (0, -1), (2, 1), (1, -2)]
        a1_of_item = {1: cw1, 2: ccw1, 3: cw2}

        def mk(i, h, j):
            slot = items[i][0]
            if h == 0:
                cw_src = colbuf.at[slot, j * SUBR:(j + 1) * SUBR, :]
                ccw_src = colbuf.at[
                    slot, HALF + j * SUBR:HALF + (j + 1) * SUBR, :]
            else:
                s = 1 if h == 1 else 0
                cw_src = cwbuf.at[i, s, j]
                ccw_src = ccwbuf.at[i, s, j]
            d = 1 if h in (0, 2) else 0
            cw = pltpu.make_async_remote_copy(
                src_ref=cw_src, dst_ref=cwbuf.at[i, d, j],
                send_sem=cw_send.at[i, h, j],
                recv_sem=cw_recv.at[i, h, j],
                device_id=(pright,), device_id_type=pl.DeviceIdType.MESH,
            )
            ccw = pltpu.make_async_remote_copy(
                src_ref=ccw_src, dst_ref=ccwbuf.at[i, d, j],
                send_sem=ccw_send.at[i, h, j],
                recv_sem=ccw_recv.at[i, h, j],
                device_id=(pleft,), device_id_type=pl.DeviceIdType.MESH,
            )
            return cw, ccw

        allhops = [[[mk(i, h, j) for j in range(SUB)] for h in range(3)]
                   for i in range(4)]

        def a1_ready(i):
            slot, dz = items[i]
            a1_of_item[i].wait_recv()
            if i == 1:
                cw2.start()
            od = lax.rem(my + dz * 4 + 2 * N_DEV, N_DEV)
            return gemm(colbuf[slot], od * m_per, m_per)

        for j in range(SUB):
            allhops[0][0][j][0].start()
            allhops[0][0][j][1].start()

        for i, (slot, dz) in enumerate(items):
            hops = allhops[i]
            for h in range(3):
                for j in range(SUB):
                    cw, ccw = hops[h][j]
                    cw.wait_recv()
                    ccw.wait_recv()
                    if h + 1 < 3:
                        hops[h + 1][j][0].start()
                        hops[h + 1][j][1].start()
                if h == 1 and i + 1 < 4:
                    local_amax = jnp.maximum(local_amax, a1_ready(i + 1))
                    for j in range(SUB):
                        allhops[i + 1][0][j][0].start()
                        allhops[i + 1][0][j][1].start()
                d = 1 if h in (0, 2) else 0
                ocw = base + lax.rem(r - h - 1 + 8, 4)
                occw = base + lax.rem(r + h + 1, 4)
                ocw = lax.rem(ocw + dz * 4 + 2 * N_DEV, N_DEV)
                occw = lax.rem(occw + dz * 4 + 2 * N_DEV, N_DEV)
                for j in range(SUB):
                    a1 = gemm(cwbuf[i, d, j],
                              ocw * m_per + j * SUBR, SUBR)
                    a2 = gemm(ccwbuf[i, d, j],
                              occw * m_per + HALF + j * SUBR, SUBR)
                    local_amax = jnp.maximum(local_amax, jnp.maximum(a1, a2))

        for i in range(4):
            for h in range(3):
                for j in range(SUB):
                    allhops[i][h][j][0].wait_send()
                    allhops[i][h][j][1].wait_send()
        cw1.wait_send()
        cw2.wait_send()
        ccw1.wait_send()

        amax_ref[N_DEV - 1, :, :] = jnp.full((8, 128), local_amax, jnp.float32)
        sends = []
        for d in range(1, N_DEV):
            tgt = lax.rem(my + d, N_DEV)
            c = pltpu.make_async_remote_copy(
                src_ref=amax_ref.at[N_DEV - 1],
                dst_ref=amax_ref.at[d - 1],
                send_sem=ax_send.at[d - 1],
                recv_sem=ax_recv.at[d - 1],
                device_id=(tgt,),
                device_id_type=pl.DeviceIdType.MESH,
            )
            c.start()
            sends.append(c)
        g = local_amax
        for d, c in zip(range(1, N_DEV), sends):
            c.wait_send()
            c.wait_recv()
            g = jnp.maximum(g, amax_ref[d - 1, 0, 0])

        scale = g / 127.0
        q = jnp.clip(jnp.round(out_ref[...] / scale), -127.0, 127.0)
        out_ref[...] = q * scale

    return pl.pallas_call(
        body,
        out_shape=jax.ShapeDtypeStruct((N_DEV * m_per, n_per), jnp.float32),
        in_specs=[
            pl.BlockSpec(memory_space=pltpu.VMEM),
            pl.BlockSpec(memory_space=pltpu.VMEM),
        ],
        out_specs=pl.BlockSpec(memory_space=pltpu.VMEM),
        scratch_shapes=[
            pltpu.VMEM((4, m_per, k), jnp.bfloat16),
            pltpu.VMEM((4, 2, SUB, SUBR, k), jnp.bfloat16),
            pltpu.VMEM((4, 2, SUB, SUBR, k), jnp.bfloat16),
            pltpu.VMEM((k, n_per), jnp.bfloat16),
            pltpu.VMEM((N_DEV, 8, 128), jnp.float32),
            pltpu.SemaphoreType.DMA((3,)),
            pltpu.SemaphoreType.DMA((3,)),
            pltpu.SemaphoreType.DMA((4, 3, SUB)),
            pltpu.SemaphoreType.DMA((4, 3, SUB)),
            pltpu.SemaphoreType.DMA((4, 3, SUB)),
            pltpu.SemaphoreType.DMA((4, 3, SUB)),
            pltpu.SemaphoreType.DMA((N_DEV - 1,)),
            pltpu.SemaphoreType.DMA((N_DEV - 1,)),
        ],
        compiler_params=pltpu.CompilerParams(collective_id=0),
    )(x, w_mat)


# device time: 138704 ns/iter; 1.3612x vs baseline; 1.1128x over previous
import jax
import jax.numpy as jnp
from jax import lax
from jax.experimental import pallas as pl
from jax.experimental.pallas import tpu as pltpu

N_DEV = 16
SUB = 2
ALPHA = 192
BETA = 64
HALF = ALPHA // 2
SUBR = HALF // SUB
BH = BETA // 2


def kernel(x, w_mat):
    m_per, k = x.shape
    _, n_per = w_mat.shape

    def body(x_ref, w_ref, out_ref, colbuf, cwbuf, ccwbuf, aggsend, aggrecv,
             wb_ref, amax_ref,
             a1_send, a1_recv, cw_send, cw_recv, ccw_send, ccw_recv,
             b1cw_send, b1cw_recv, b1ccw_send, b1ccw_recv, b2_send, b2_recv,
             ax_send, ax_recv):
        my = lax.axis_index("i")
        r = lax.rem(my, 4)
        base = my - r
        up = lax.rem(my + 4, N_DEV)
        down = lax.rem(my + 12, N_DEV)
        pright = base + lax.rem(r + 1, 4)
        pleft = base + lax.rem(r + 3, 4)

        barrier_sem = pltpu.get_barrier_semaphore()
        for nbr in (up, down, pright, pleft):
            pl.semaphore_signal(
                barrier_sem, inc=1,
                device_id=(nbr,), device_id_type=pl.DeviceIdType.MESH,
            )
        pl.semaphore_wait(barrier_sem, 4)

        wb_ref[...] = w_ref[...].astype(jnp.bfloat16)
        xb = x_ref[...].astype(jnp.bfloat16)
        colbuf[3, :, :] = xb[:ALPHA, :]
        aggsend[0, :, :] = xb[ALPHA:, :]

        def col_copy(src, dst, sems, si, dev):
            return pltpu.make_async_remote_copy(
                src_ref=src, dst_ref=dst,
                send_sem=sems[0].at[si], recv_sem=sems[1].at[si],
                device_id=(dev,), device_id_type=pl.DeviceIdType.MESH,
            )

        cw1 = col_copy(colbuf.at[3], colbuf.at[0], (a1_send, a1_recv), 0, up)
        cw2 = col_copy(colbuf.at[0], colbuf.at[1], (a1_send, a1_recv), 1, up)
        ccw1 = col_copy(colbuf.at[3], colbuf.at[2], (a1_send, a1_recv), 2,
                        down)
        cw1.start()
        ccw1.start()

        def gemm(chunk, row0, rows):
            y = lax.dot_general(
                chunk, wb_ref[...], (((1,), (0,)), ((), ())),
                preferred_element_type=jnp.float32,
            )
            y = jnp.maximum(y, 0.0)
            out_ref[pl.ds(row0, rows), :] = y
            return jnp.max(y)

        b1cw = []
        b1ccw = []
        for h in range(3):
            scw = 0 if h == 0 else 4 - h
            sccw = 0 if h == 0 else h
            b1cw.append(pltpu.make_async_remote_copy(
                src_ref=aggsend.at[scw, 0:BH, :],
                dst_ref=aggsend.at[3 - h, 0:BH, :],
                send_sem=b1cw_send.at[h], recv_sem=b1cw_recv.at[h],
                device_id=(pright,), device_id_type=pl.DeviceIdType.MESH,
            ))
            b1ccw.append(pltpu.make_async_remote_copy(
                src_ref=aggsend.at[sccw, BH:2 * BH, :],
                dst_ref=aggsend.at[h + 1, BH:2 * BH, :],
                send_sem=b1ccw_send.at[h], recv_sem=b1ccw_recv.at[h],
                device_id=(pleft,), device_id_type=pl.DeviceIdType.MESH,
            ))

        b2cw1 = col_copy(aggsend, aggrecv.at[0], (b2_send, b2_recv), 0, up)
        b2cw2 = col_copy(aggrecv.at[0], aggrecv.at[1], (b2_send, b2_recv),
                         1, up)
        b2ccw1 = col_copy(aggsend, aggrecv.at[2], (b2_send, b2_recv), 2,
                          down)

        def b2_gemms(slot, src_base, amax):
            for dd in range(4):
                od = lax.rem(src_base + lax.rem(r + dd, 4), N_DEV)
                amax = jnp.maximum(
                    amax,
                    gemm(aggrecv[slot, dd], od * m_per + ALPHA, BETA))
            return amax

        items = [(3, 0), (0, -1), (2, 1), (1, -2)]
        a1_of_item = {1: cw1, 2: ccw1, 3: cw2}

        def mk(i, h, j):
            slot = items[i][0]
            if h == 0:
                cw_src = colbuf.at[slot, j * SUBR:(j + 1) * SUBR, :]
                ccw_src = colbuf.at[
                    slot, HALF + j * SUBR:HALF + (j + 1) * SUBR, :]
            else:
                s = 1 if h == 1 else 0
                cw_src = cwbuf.at[i, s, j]
                ccw_src = ccwbuf.at[i, s, j]
            d = 1 if h in (0, 2) else 0
            cw = pltpu.make_async_remote_copy(
                src_ref=cw_src, dst_ref=cwbuf.at[i, d, j],
                send_sem=cw_send.at[i, h, j],
                recv_sem=cw_recv.at[i, h, j],
                device_id=(pright,), device_id_type=pl.DeviceIdType.MESH,
            )
            ccw = pltpu.make_async_remote_copy(
                src_ref=ccw_src, dst_ref=ccwbuf.at[i, d, j],
                send_sem=ccw_send.at[i, h, j],
                recv_sem=ccw_recv.at[i, h, j],
                device_id=(pleft,), device_id_type=pl.DeviceIdType.MESH,
            )
            return cw, ccw

        allhops = [[[mk(i, h, j) for j in range(SUB)] for h in range(3)]
                   for i in range(4)]

        def a1_ready(i):
            slot, dz = items[i]
            a1_of_item[i].wait_recv()
            if i == 1:
                cw2.start()
            od = lax.rem(my + dz * 4 + 2 * N_DEV, N_DEV)
            return gemm(colbuf[slot], od * m_per, ALPHA)

        for j in range(SUB):
            allhops[0][0][j][0].start()
            allhops[0][0][j][1].start()
        b1cw[0].start()
        b1ccw[0].start()

        local_amax = gemm(xb, my * m_per, m_per)

        for i, (slot, dz) in enumerate(items):
            hops = allhops[i]
            for h in range(3):
                for j in range(SUB):
                    cw, ccw = hops[h][j]
                    cw.wait_recv()
                    ccw.wait_recv()
                    if h + 1 < 3:
                        hops[h + 1][j][0].start()
                        hops[h + 1][j][1].start()
                if i == 0:
                    b1cw[h].wait_recv()
                    b1ccw[h].wait_recv()
                    if h + 1 < 3:
                        b1cw[h + 1].start()
                        b1ccw[h + 1].start()
                    else:
                        b2cw1.start()
                        b2ccw1.start()
                    ob1 = lax.rem(base + lax.rem(r - h - 1 + 8, 4), N_DEV)
                    ob2 = lax.rem(base + lax.rem(r + h + 1, 4), N_DEV)
                    a1 = gemm(aggsend[3 - h, 0:BH, :],
                              ob1 * m_per + ALPHA, BH)
                    a2 = gemm(aggsend[h + 1, BH:2 * BH, :],
                              ob2 * m_per + ALPHA + BH, BH)
                    local_amax = jnp.maximum(local_amax, jnp.maximum(a1, a2))
                if h == 1 and i + 1 < 4:
                    local_amax = jnp.maximum(local_amax, a1_ready(i + 1))
                    for j in range(SUB):
                        allhops[i + 1][0][j][0].start()
                        allhops[i + 1][0][j][1].start()
                if i == 2 and h == 1:
                    b2cw1.wait_recv()
                    b2cw2.start()
                    local_amax = b2_gemms(0, base + 12, local_amax)
                if i == 3 and h == 0:
                    b2ccw1.wait_recv()
                    local_amax = b2_gemms(2, base + 4, local_amax)
                d = 1 if h in (0, 2) else 0
                ocw = base + lax.rem(r - h - 1 + 8, 4)
                occw = base + lax.rem(r + h + 1, 4)
                ocw = lax.rem(ocw + dz * 4 + 2 * N_DEV, N_DEV)
                occw = lax.rem(occw + dz * 4 + 2 * N_DEV, N_DEV)
                for j in range(SUB):
                    a1 = gemm(cwbuf[i, d, j],
                              ocw * m_per + j * SUBR, SUBR)
                    a2 = gemm(ccwbuf[i, d, j],
                              occw * m_per + HALF + j * SUBR, SUBR)
                    local_amax = jnp.maximum(local_amax, jnp.maximum(a1, a2))

        b2cw2.wait_recv()
        local_amax = b2_gemms(1, base + 8, local_amax)

        for i in range(4):
            for h in range(3):
                for j in range(SUB):
                    allhops[i][h][j][0].wait_send()
                    allhops[i][h][j][1].wait_send()
        for h in range(3):
            b1cw[h].wait_send()
            b1ccw[h].wait_send()
        cw1.wait_send()
        cw2.wait_send()
        ccw1.wait_send()
        b2cw1.wait_send()
        b2cw2.wait_send()
        b2ccw1.wait_send()

        amax_ref[N_DEV - 1, :, :] = jnp.full((8, 128), local_amax, jnp.float32)
        sends = []
        for d in range(1, N_DEV):
            tgt = lax.rem(my + d, N_DEV)
            c = pltpu.make_async_remote_copy(
                src_ref=amax_ref.at[N_DEV - 1],
                dst_ref=amax_ref.at[d - 1],
                send_sem=ax_send.at[d - 1],
                recv_sem=ax_recv.at[d - 1],
                device_id=(tgt,),
                device_id_type=pl.DeviceIdType.MESH,
            )
            c.start()
            sends.append(c)
        g = local_amax
        for d, c in zip(range(1, N_DEV), sends):
            c.wait_send()
            c.wait_recv()
            g = jnp.maximum(g, amax_ref[d - 1, 0, 0])

        scale = g / 127.0
        q = jnp.clip(jnp.round(out_ref[...] / scale), -127.0, 127.0)
        out_ref[...] = q * scale

    return pl.pallas_call(
        body,
        out_shape=jax.ShapeDtypeStruct((N_DEV * m_per, n_per), jnp.float32),
        in_specs=[
            pl.BlockSpec(memory_space=pltpu.VMEM),
            pl.BlockSpec(memory_space=pltpu.VMEM),
        ],
        out_specs=pl.BlockSpec(memory_space=pltpu.VMEM),
        scratch_shapes=[
            pltpu.VMEM((4, ALPHA, k), jnp.bfloat16),
            pltpu.VMEM((4, 2, SUB, SUBR, k), jnp.bfloat16),
            pltpu.VMEM((4, 2, SUB, SUBR, k), jnp.bfloat16),
            pltpu.VMEM((4, BETA, k), jnp.bfloat16),
            pltpu.VMEM((3, 4, BETA, k), jnp.bfloat16),
            pltpu.VMEM((k, n_per), jnp.bfloat16),
            pltpu.VMEM((N_DEV, 8, 128), jnp.float32),
            pltpu.SemaphoreType.DMA((3,)),
            pltpu.SemaphoreType.DMA((3,)),
            pltpu.SemaphoreType.DMA((4, 3, SUB)),
            pltpu.SemaphoreType.DMA((4, 3, SUB)),
            pltpu.SemaphoreType.DMA((4, 3, SUB)),
            pltpu.SemaphoreType.DMA((4, 3, SUB)),
            pltpu.SemaphoreType.DMA((3,)),
            pltpu.SemaphoreType.DMA((3,)),
            pltpu.SemaphoreType.DMA((3,)),
            pltpu.SemaphoreType.DMA((3,)),
            pltpu.SemaphoreType.DMA((3,)),
            pltpu.SemaphoreType.DMA((3,)),
            pltpu.SemaphoreType.DMA((N_DEV - 1,)),
            pltpu.SemaphoreType.DMA((N_DEV - 1,)),
        ],
        compiler_params=pltpu.CompilerParams(collective_id=0),
    )(x, w_mat)


# device time: 138507 ns/iter; 1.3631x vs baseline; 1.0014x over previous
import jax
import jax.numpy as jnp
from jax import lax
from jax.experimental import pallas as pl
from jax.experimental.pallas import tpu as pltpu

N_DEV = 16
SUB = 2
ALPHA = 192
BETA = 64
HALF = ALPHA // 2
SUBR = HALF // SUB
BH = BETA // 2


def kernel(x, w_mat):
    m_per, k = x.shape
    _, n_per = w_mat.shape

    def body(x_ref, w_ref, out_ref, colbuf, cwbuf, ccwbuf, aggsend, aggrecv,
             wb_ref, amax_ref,
             a1_send, a1_recv, cw_send, cw_recv, ccw_send, ccw_recv,
             b1cw_send, b1cw_recv, b1ccw_send, b1ccw_recv, b2_send, b2_recv,
             ax_send, ax_recv):
        my = lax.axis_index("i")
        r = lax.rem(my, 4)
        base = my - r
        up = lax.rem(my + 4, N_DEV)
        down = lax.rem(my + 12, N_DEV)
        pright = base + lax.rem(r + 1, 4)
        pleft = base + lax.rem(r + 3, 4)

        barrier_sem = pltpu.get_barrier_semaphore()
        for nbr in (up, down, pright, pleft):
            pl.semaphore_signal(
                barrier_sem, inc=1,
                device_id=(nbr,), device_id_type=pl.DeviceIdType.MESH,
            )
        pl.semaphore_wait(barrier_sem, 4)

        xb = x_ref[...].astype(jnp.bfloat16)
        colbuf[3, :, :] = xb[:ALPHA, :]

        def col_copy(src, dst, sems, si, dev):
            return pltpu.make_async_remote_copy(
                src_ref=src, dst_ref=dst,
                send_sem=sems[0].at[si], recv_sem=sems[1].at[si],
                device_id=(dev,), device_id_type=pl.DeviceIdType.MESH,
            )

        cw1 = col_copy(colbuf.at[3], colbuf.at[0], (a1_send, a1_recv), 0, up)
        cw2 = col_copy(colbuf.at[0], colbuf.at[1], (a1_send, a1_recv), 1, up)
        ccw1 = col_copy(colbuf.at[3], colbuf.at[2], (a1_send, a1_recv), 2,
                        down)
        cw1.start()
        ccw1.start()
        aggsend[0, :, :] = xb[ALPHA:, :]

        def gemm(chunk, row0, rows):
            y = lax.dot_general(
                chunk, wb_ref[...], (((1,), (0,)), ((), ())),
                preferred_element_type=jnp.float32,
            )
            y = jnp.maximum(y, 0.0)
            out_ref[pl.ds(row0, rows), :] = y
            return jnp.max(y)

        b1cw = []
        b1ccw = []
        for h in range(3):
            scw = 0 if h == 0 else 4 - h
            sccw = 0 if h == 0 else h
            b1cw.append(pltpu.make_async_remote_copy(
                src_ref=aggsend.at[scw, 0:BH, :],
                dst_ref=aggsend.at[3 - h, 0:BH, :],
                send_sem=b1cw_send.at[h], recv_sem=b1cw_recv.at[h],
                device_id=(pright,), device_id_type=pl.DeviceIdType.MESH,
            ))
            b1ccw.append(pltpu.make_async_remote_copy(
                src_ref=aggsend.at[sccw, BH:2 * BH, :],
                dst_ref=aggsend.at[h + 1, BH:2 * BH, :],
                send_sem=b1ccw_send.at[h], recv_sem=b1ccw_recv.at[h],
                device_id=(pleft,), device_id_type=pl.DeviceIdType.MESH,
            ))

        b2cw1 = col_copy(aggsend, aggrecv.at[0], (b2_send, b2_recv), 0, up)
        b2cw2 = col_copy(aggrecv.at[0], aggrecv.at[1], (b2_send, b2_recv),
                         1, up)
        b2ccw1 = col_copy(aggsend, aggrecv.at[2], (b2_send, b2_recv), 2,
                          down)

        def b2_gemms(slot, src_base, amax):
            for dd in range(4):
                od = lax.rem(src_base + lax.rem(r + dd, 4), N_DEV)
                amax = jnp.maximum(
                    amax,
                    gemm(aggrecv[slot, dd], od * m_per + ALPHA, BETA))
            return amax

        items = [(3, 0), (0, -1), (2, 1), (1, -2)]
        a1_of_item = {1: cw1, 2: ccw1, 3: cw2}

        def mk(i, h, j):
            slot = items[i][0]
            if h == 0:
                cw_src = colbuf.at[slot, j * SUBR:(j + 1) * SUBR, :]
                ccw_src = colbuf.at[
                    slot, HALF + j * SUBR:HALF + (j + 1) * SUBR, :]
            else:
                s = 1 if h == 1 else 0
                cw_src = cwbuf.at[i, s, j]
                ccw_src = ccwbuf.at[i, s, j]
            d = 1 if h in (0, 2) else 0
            cw = pltpu.make_async_remote_copy(
                src_ref=cw_src, dst_ref=cwbuf.at[i, d, j],
                send_sem=cw_send.at[i, h, j],
                recv_sem=cw_recv.at[i, h, j],
                device_id=(pright,), device_id_type=pl.DeviceIdType.MESH,
            )
            ccw = pltpu.make_async_remote_copy(
                src_ref=ccw_src, dst_ref=ccwbuf.at[i, d, j],
                send_sem=ccw_send.at[i, h, j],
                recv_sem=ccw_recv.at[i, h, j],
                device_id=(pleft,), device_id_type=pl.DeviceIdType.MESH,
            )
            return cw, ccw

        allhops = [[[mk(i, h, j) for j in range(SUB)] for h in range(3)]
                   for i in range(4)]

        def a1_ready(i):
            slot, dz = items[i]
            a1_of_item[i].wait_recv()
            if i == 1:
                cw2.start()
            od = lax.rem(my + dz * 4 + 2 * N_DEV, N_DEV)
            return gemm(colbuf[slot], od * m_per, ALPHA)

        for j in range(SUB):
            allhops[0][0][j][0].start()
            allhops[0][0][j][1].start()
        b1cw[0].start()
        b1ccw[0].start()

        wb_ref[...] = w_ref[...].astype(jnp.bfloat16)
        local_amax = gemm(xb, my * m_per, m_per)

        for i, (slot, dz) in enumerate(items):
            hops = allhops[i]
            for h in range(3):
                for j in range(SUB):
                    cw, ccw = hops[h][j]
                    cw.wait_recv()
                    ccw.wait_recv()
                    if h + 1 < 3:
                        hops[h + 1][j][0].start()
                        hops[h + 1][j][1].start()
                if i == 0:
                    b1cw[h].wait_recv()
                    b1ccw[h].wait_recv()
                    if h + 1 < 3:
                        b1cw[h + 1].start()
                        b1ccw[h + 1].start()
                    else:
                        b2cw1.start()
                        b2ccw1.start()
                    ob1 = lax.rem(base + lax.rem(r - h - 1 + 8, 4), N_DEV)
                    ob2 = lax.rem(base + lax.rem(r + h + 1, 4), N_DEV)
                    a1 = gemm(aggsend[3 - h, 0:BH, :],
                              ob1 * m_per + ALPHA, BH)
                    a2 = gemm(aggsend[h + 1, BH:2 * BH, :],
                              ob2 * m_per + ALPHA + BH, BH)
                    local_amax = jnp.maximum(local_amax, jnp.maximum(a1, a2))
                if h == 1 and i + 1 < 4:
                    local_amax = jnp.maximum(local_amax, a1_ready(i + 1))
                    for j in range(SUB):
                        allhops[i + 1][0][j][0].start()
                        allhops[i + 1][0][j][1].start()
                if i == 2 and h == 1:
                    b2cw1.wait_recv()
                    b2cw2.start()
                    local_amax = b2_gemms(0, base + 12, local_amax)
                if i == 3 and h == 0:
                    b2ccw1.wait_recv()
                    local_amax = b2_gemms(2, base + 4, local_amax)
                d = 1 if h in (0, 2) else 0
                ocw = base + lax.rem(r - h - 1 + 8, 4)
                occw = base + lax.rem(r + h + 1, 4)
                ocw = lax.rem(ocw + dz * 4 + 2 * N_DEV, N_DEV)
                occw = lax.rem(occw + dz * 4 + 2 * N_DEV, N_DEV)
                for j in range(SUB):
                    a1 = gemm(cwbuf[i, d, j],
                              ocw * m_per + j * SUBR, SUBR)
                    a2 = gemm(ccwbuf[i, d, j],
                              occw * m_per + HALF + j * SUBR, SUBR)
                    local_amax = jnp.maximum(local_amax, jnp.maximum(a1, a2))

        b2cw2.wait_recv()
        local_amax = b2_gemms(1, base + 8, local_amax)

        for i in range(4):
            for h in range(3):
                for j in range(SUB):
                    allhops[i][h][j][0].wait_send()
                    allhops[i][h][j][1].wait_send()
        for h in range(3):
            b1cw[h].wait_send()
            b1ccw[h].wait_send()
        cw1.wait_send()
        cw2.wait_send()
        ccw1.wait_send()
        b2cw1.wait_send()
        b2cw2.wait_send()
        b2ccw1.wait_send()

        amax_ref[N_DEV - 1, :, :] = jnp.full((8, 128), local_amax, jnp.float32)
        sends = []
        for d in range(1, N_DEV):
            tgt = lax.rem(my + d, N_DEV)
            c = pltpu.make_async_remote_copy(
                src_ref=amax_ref.at[N_DEV - 1],
                dst_ref=amax_ref.at[d - 1],
                send_sem=ax_send.at[d - 1],
                recv_sem=ax_recv.at[d - 1],
                device_id=(tgt,),
                device_id_type=pl.DeviceIdType.MESH,
            )
            c.start()
            sends.append(c)
        g = local_amax
        for d, c in zip(range(1, N_DEV), sends):
            c.wait_send()
            c.wait_recv()
            g = jnp.maximum(g, amax_ref[d - 1, 0, 0])

        scale = g / 127.0
        q = jnp.clip(jnp.round(out_ref[...] / scale), -127.0, 127.0)
        out_ref[...] = q * scale

    return pl.pallas_call(
        body,
        out_shape=jax.ShapeDtypeStruct((N_DEV * m_per, n_per), jnp.float32),
        in_specs=[
            pl.BlockSpec(memory_space=pltpu.VMEM),
            pl.BlockSpec(memory_space=pltpu.VMEM),
        ],
        out_specs=pl.BlockSpec(memory_space=pltpu.VMEM),
        scratch_shapes=[
            pltpu.VMEM((4, ALPHA, k), jnp.bfloat16),
            pltpu.VMEM((4, 2, SUB, SUBR, k), jnp.bfloat16),
            pltpu.VMEM((4, 2, SUB, SUBR, k), jnp.bfloat16),
            pltpu.VMEM((4, BETA, k), jnp.bfloat16),
            pltpu.VMEM((3, 4, BETA, k), jnp.bfloat16),
            pltpu.VMEM((k, n_per), jnp.bfloat16),
            pltpu.VMEM((N_DEV, 8, 128), jnp.float32),
            pltpu.SemaphoreType.DMA((3,)),
            pltpu.SemaphoreType.DMA((3,)),
            pltpu.SemaphoreType.DMA((4, 3, SUB)),
            pltpu.SemaphoreType.DMA((4, 3, SUB)),
            pltpu.SemaphoreType.DMA((4, 3, SUB)),
            pltpu.SemaphoreType.DMA((4, 3, SUB)),
            pltpu.SemaphoreType.DMA((3,)),
            pltpu.SemaphoreType.DMA((3,)),
            pltpu.SemaphoreType.DMA((3,)),
            pltpu.SemaphoreType.DMA((3,)),
            pltpu.SemaphoreType.DMA((3,)),
            pltpu.SemaphoreType.DMA((3,)),
            pltpu.SemaphoreType.DMA((N_DEV - 1,)),
            pltpu.SemaphoreType.DMA((N_DEV - 1,)),
        ],
        compiler_params=pltpu.CompilerParams(collective_id=0),
    )(x, w_mat)
